# baseline (speedup 1.0000x reference)
import functools

import jax
import jax.numpy as jnp
from jax import lax
from jax.experimental import pallas as pl
from jax.experimental.pallas import tpu as pltpu
from jax.experimental.pallas import tpu_sc as plsc

_LANES = 16


@functools.lru_cache(maxsize=None)
def _make_sc_scale_kernel(rows: int, dim: int, scale: float):
    info = plsc.get_sparse_core_info()
    num_workers = info.num_cores * info.num_subcores
    assert rows % num_workers == 0
    rows_per_worker = rows // num_workers
    chunk_rows = min(rows_per_worker, 32)
    n_chunks = rows_per_worker // chunk_rows

    mesh = plsc.VectorSubcoreMesh(core_axis_name="c", subcore_axis_name="s")

    @functools.partial(
        pl.kernel,
        mesh=mesh,
        out_type=jax.ShapeDtypeStruct((rows, dim), jnp.float32),
        scratch_types=[
            pltpu.VMEM((chunk_rows, dim), jnp.float32),
            pltpu.VMEM((chunk_rows, dim), jnp.float32),
            pltpu.SemaphoreType.DMA,
            pltpu.SemaphoreType.DMA,
            pltpu.SemaphoreType.DMA,
            pltpu.SemaphoreType.DMA,
        ],
        compiler_params=pltpu.CompilerParams(use_tc_tiling_on_sc=True),
    )
    def sc_scale(emb_hbm, out_hbm, buf0, buf1, si0, si1, so0, so1):
        wid = lax.axis_index("s") * info.num_cores + lax.axis_index("c")
        base = wid * rows_per_worker
        bufs = (buf0, buf1)
        sin = (si0, si1)
        sout = (so0, so1)

        in_copies = [None, None]
        out_copies = [None, None]
        in_copies[0] = pltpu.async_copy(
            emb_hbm.at[pl.ds(base, chunk_rows)], buf0, si0)
        for c in range(n_chunks):
            b = c % 2
            nb = (c + 1) % 2
            if c + 1 < n_chunks:
                if out_copies[nb] is not None:
                    out_copies[nb].wait()
                in_copies[nb] = pltpu.async_copy(
                    emb_hbm.at[pl.ds(base + (c + 1) * chunk_rows, chunk_rows)],
                    bufs[nb], sin[nb])
            in_copies[b].wait()
            buf = bufs[b]

            def row_body(r, _):
                @plsc.parallel_loop(0, dim, step=_LANES, unroll=8)
                def _scale(i):
                    buf[r, pl.ds(i, _LANES)] = buf[r, pl.ds(i, _LANES)] * scale

                return 0

            lax.fori_loop(0, chunk_rows, row_body, 0)

            out_copies[b] = pltpu.async_copy(
                buf, out_hbm.at[pl.ds(base + c * chunk_rows, chunk_rows)],
                sout[b])
        if n_chunks > 1:
            out_copies[(n_chunks - 2) % 2].wait()
        out_copies[(n_chunks - 1) % 2].wait()

    return sc_scale


@functools.lru_cache(maxsize=None)
def _make_tc_scale_kernel(row_start: int, rows: int, dim: int, scale: float):
    block_rows = 256
    assert rows % block_rows == 0 and row_start % block_rows == 0
    n_blocks = rows // block_rows
    first_block = row_start // block_rows

    def tc_scale(e_ref, o_ref):
        o_ref[...] = e_ref[...] * scale

    return pl.pallas_call(
        tc_scale,
        grid=(n_blocks,),
        in_specs=[pl.BlockSpec((block_rows, dim),
                               lambda i: (first_block + i, 0))],
        out_specs=pl.BlockSpec((block_rows, dim), lambda i: (i, 0)),
        out_shape=jax.ShapeDtypeStruct((rows, dim), jnp.float32),
    )


def kernel(x, emb_weight):
    seq_len = x.shape[1]
    dim = emb_weight.shape[1]
    scale = dim ** -0.5
    sc_rows = (3 * seq_len // 4) // 256 * 256
    sc_out = _make_sc_scale_kernel(sc_rows, dim, scale)(emb_weight)
    tc_out = _make_tc_scale_kernel(sc_rows, seq_len - sc_rows, dim, scale)(
        emb_weight)
    return jnp.concatenate([sc_out, tc_out], axis=0)


# speedup vs baseline: 1.3824x; 1.3824x over previous
import functools

import jax
import jax.numpy as jnp
from jax import lax
from jax.experimental import pallas as pl
from jax.experimental.pallas import tpu as pltpu
from jax.experimental.pallas import tpu_sc as plsc

_LANES = 16


@functools.lru_cache(maxsize=None)
def _make_scale_kernel(rows: int, dim: int, scale: float):
    info = plsc.get_sparse_core_info()
    num_workers = info.num_cores * info.num_subcores
    assert rows % num_workers == 0
    rows_per_worker = rows // num_workers
    chunk_rows = min(rows_per_worker, 16)
    n_chunks = rows_per_worker // chunk_rows

    mesh = plsc.VectorSubcoreMesh(core_axis_name="c", subcore_axis_name="s")

    @functools.partial(
        pl.kernel,
        mesh=mesh,
        out_type=jax.ShapeDtypeStruct((rows, dim), jnp.float32),
        scratch_types=[
            pltpu.VMEM((chunk_rows, dim), jnp.float32),
            pltpu.VMEM((chunk_rows, dim), jnp.float32),
            pltpu.SemaphoreType.DMA,
            pltpu.SemaphoreType.DMA,
            pltpu.SemaphoreType.DMA,
            pltpu.SemaphoreType.DMA,
        ],
        compiler_params=pltpu.CompilerParams(use_tc_tiling_on_sc=True),
    )
    def scale_kernel(emb_hbm, out_hbm, buf0, buf1, si0, si1, so0, so1):
        wid = lax.axis_index("s") * info.num_cores + lax.axis_index("c")
        base = wid * rows_per_worker
        bufs = (buf0, buf1)
        sin = (si0, si1)
        sout = (so0, so1)

        in_copies = [None, None]
        out_copies = [None, None]
        in_copies[0] = pltpu.async_copy(
            emb_hbm.at[pl.ds(base, chunk_rows)], buf0, si0)
        for c in range(n_chunks):
            b = c % 2
            nb = (c + 1) % 2
            if c + 1 < n_chunks:
                if out_copies[nb] is not None:
                    out_copies[nb].wait()
                in_copies[nb] = pltpu.async_copy(
                    emb_hbm.at[pl.ds(base + (c + 1) * chunk_rows, chunk_rows)],
                    bufs[nb], sin[nb])
            in_copies[b].wait()
            buf = bufs[b]

            def row_body(r, _):
                @plsc.parallel_loop(0, dim, step=_LANES, unroll=8)
                def _scale(i):
                    buf[r, pl.ds(i, _LANES)] = buf[r, pl.ds(i, _LANES)] * scale

                return 0

            lax.fori_loop(0, chunk_rows, row_body, 0)

            out_copies[b] = pltpu.async_copy(
                buf, out_hbm.at[pl.ds(base + c * chunk_rows, chunk_rows)],
                sout[b])
        if n_chunks > 1:
            out_copies[(n_chunks - 2) % 2].wait()
        out_copies[(n_chunks - 1) % 2].wait()

    return scale_kernel


def kernel(x, emb_weight):
    seq_len = x.shape[1]
    dim = emb_weight.shape[1]
    scale = dim ** -0.5
    return _make_scale_kernel(seq_len, dim, scale)(emb_weight[:seq_len])


# speedup vs baseline: 1.4348x; 1.0380x over previous
import functools

import jax
import jax.numpy as jnp
from jax import lax
from jax.experimental import pallas as pl
from jax.experimental.pallas import tpu as pltpu
from jax.experimental.pallas import tpu_sc as plsc

_LANES = 16


@functools.lru_cache(maxsize=None)
def _make_scale_kernel(rows: int, dim: int, scale: float):
    info = plsc.get_sparse_core_info()
    num_workers = info.num_cores * info.num_subcores
    assert rows % num_workers == 0
    rows_per_worker = rows // num_workers
    chunk_rows = min(rows_per_worker, 32)
    n_chunks = rows_per_worker // chunk_rows

    mesh = plsc.VectorSubcoreMesh(core_axis_name="c", subcore_axis_name="s")

    @functools.partial(
        pl.kernel,
        mesh=mesh,
        out_type=jax.ShapeDtypeStruct((rows, dim), jnp.float32),
        scratch_types=[
            pltpu.VMEM((chunk_rows, dim), jnp.float32),
            pltpu.VMEM((chunk_rows, dim), jnp.float32),
            pltpu.SemaphoreType.DMA,
            pltpu.SemaphoreType.DMA,
            pltpu.SemaphoreType.DMA,
            pltpu.SemaphoreType.DMA,
        ],
        compiler_params=pltpu.CompilerParams(
            use_tc_tiling_on_sc=True,
            disable_bounds_checks=True,
            skip_device_barrier=True,
        ),
    )
    def scale_kernel(emb_hbm, out_hbm, buf0, buf1, si0, si1, so0, so1):
        wid = lax.axis_index("s") * info.num_cores + lax.axis_index("c")
        base = wid * rows_per_worker
        bufs = (buf0, buf1)
        sin = (si0, si1)
        sout = (so0, so1)

        in_copies = [None, None]
        out_copies = [None, None]
        in_copies[0] = pltpu.async_copy(
            emb_hbm.at[pl.ds(base, chunk_rows)], buf0, si0)
        for c in range(n_chunks):
            b = c % 2
            nb = (c + 1) % 2
            if c + 1 < n_chunks:
                if out_copies[nb] is not None:
                    out_copies[nb].wait()
                in_copies[nb] = pltpu.async_copy(
                    emb_hbm.at[pl.ds(base + (c + 1) * chunk_rows, chunk_rows)],
                    bufs[nb], sin[nb])
            in_copies[b].wait()
            buf = bufs[b]

            def row_body(r, _):
                @plsc.parallel_loop(0, dim, step=_LANES, unroll=8)
                def _scale(i):
                    buf[r, pl.ds(i, _LANES)] = buf[r, pl.ds(i, _LANES)] * scale

                return 0

            lax.fori_loop(0, chunk_rows, row_body, 0)

            out_copies[b] = pltpu.async_copy(
                buf, out_hbm.at[pl.ds(base + c * chunk_rows, chunk_rows)],
                sout[b])
        if n_chunks > 1:
            out_copies[(n_chunks - 2) % 2].wait()
        out_copies[(n_chunks - 1) % 2].wait()

    return scale_kernel


def kernel(x, emb_weight):
    seq_len = x.shape[1]
    dim = emb_weight.shape[1]
    scale = dim ** -0.5
    return _make_scale_kernel(seq_len, dim, scale)(emb_weight[:seq_len])


# speedup vs baseline: 1.4755x; 1.0283x over previous
import functools

import jax
import jax.numpy as jnp
from jax import lax
from jax.experimental import pallas as pl
from jax.experimental.pallas import tpu as pltpu
from jax.experimental.pallas import tpu_sc as plsc

_LANES = 16


@functools.lru_cache(maxsize=None)
def _make_scale_kernel(rows: int, dim: int, scale: float):
    info = plsc.get_sparse_core_info()
    num_workers = info.num_cores * info.num_subcores
    assert rows % num_workers == 0
    rows_per_worker = rows // num_workers
    chunk_rows = min(rows_per_worker, 32)
    n_chunks = rows_per_worker // chunk_rows

    mesh = plsc.VectorSubcoreMesh(core_axis_name="c", subcore_axis_name="s")

    @functools.partial(
        pl.kernel,
        mesh=mesh,
        out_type=jax.ShapeDtypeStruct((rows, dim), jnp.float32),
        scratch_types=[
            pltpu.VMEM((chunk_rows, dim), jnp.float32),
            pltpu.VMEM((chunk_rows, dim), jnp.float32),
            pltpu.VMEM((chunk_rows, dim), jnp.float32),
            pltpu.SemaphoreType.DMA,
            pltpu.SemaphoreType.DMA,
            pltpu.SemaphoreType.DMA,
            pltpu.SemaphoreType.DMA,
            pltpu.SemaphoreType.DMA,
            pltpu.SemaphoreType.DMA,
        ],
        compiler_params=pltpu.CompilerParams(
            use_tc_tiling_on_sc=True,
            disable_bounds_checks=True,
            skip_device_barrier=True,
        ),
    )
    def scale_kernel(emb_hbm, out_hbm, buf0, buf1, buf2,
                     si0, si1, si2, so0, so1, so2):
        wid = lax.axis_index("s") * info.num_cores + lax.axis_index("c")
        base = wid * rows_per_worker
        nbuf = 3
        bufs = (buf0, buf1, buf2)
        sin = (si0, si1, si2)
        sout = (so0, so1, so2)

        in_copies = [None] * nbuf
        out_copies = [None] * nbuf
        for p in range(min(2, n_chunks)):
            in_copies[p] = pltpu.async_copy(
                emb_hbm.at[pl.ds(base + p * chunk_rows, chunk_rows)],
                bufs[p], sin[p])
        for c in range(n_chunks):
            b = c % nbuf
            nb = (c + 2) % nbuf
            if c + 2 < n_chunks:
                if out_copies[nb] is not None:
                    out_copies[nb].wait()
                in_copies[nb] = pltpu.async_copy(
                    emb_hbm.at[pl.ds(base + (c + 2) * chunk_rows, chunk_rows)],
                    bufs[nb], sin[nb])
            in_copies[b].wait()
            buf = bufs[b]

            def row_body(r, _):
                @plsc.parallel_loop(0, dim, step=_LANES, unroll=8)
                def _scale(i):
                    buf[r, pl.ds(i, _LANES)] = buf[r, pl.ds(i, _LANES)] * scale

                return 0

            lax.fori_loop(0, chunk_rows, row_body, 0)

            out_copies[b] = pltpu.async_copy(
                buf, out_hbm.at[pl.ds(base + c * chunk_rows, chunk_rows)],
                sout[b])
        for p in range(min(nbuf, n_chunks)):
            out_copies[(n_chunks - 1 - p) % nbuf].wait()

    return scale_kernel


def kernel(x, emb_weight):
    seq_len = x.shape[1]
    dim = emb_weight.shape[1]
    scale = dim ** -0.5
    return _make_scale_kernel(seq_len, dim, scale)(emb_weight[:seq_len])


# speedup vs baseline: 1.4760x; 1.0003x over previous
import functools

import jax
import jax.numpy as jnp
from jax import lax
from jax.experimental import pallas as pl
from jax.experimental.pallas import tpu as pltpu
from jax.experimental.pallas import tpu_sc as plsc

_LANES = 16


@functools.lru_cache(maxsize=None)
def _make_scale_kernel(rows: int, dim: int, scale: float):
    info = plsc.get_sparse_core_info()
    num_workers = info.num_cores * info.num_subcores
    assert rows % num_workers == 0
    rows_per_worker = rows // num_workers
    chunk_rows = min(rows_per_worker, 32)
    n_chunks = rows_per_worker // chunk_rows

    mesh = plsc.VectorSubcoreMesh(core_axis_name="c", subcore_axis_name="s")

    @functools.partial(
        pl.kernel,
        mesh=mesh,
        out_type=jax.ShapeDtypeStruct((rows, dim), jnp.float32),
        scratch_types=[
            pltpu.VMEM((chunk_rows, dim), jnp.float32),
            pltpu.VMEM((chunk_rows, dim), jnp.float32),
            pltpu.VMEM((chunk_rows, dim), jnp.float32),
            pltpu.SemaphoreType.DMA,
            pltpu.SemaphoreType.DMA,
            pltpu.SemaphoreType.DMA,
            pltpu.SemaphoreType.DMA,
            pltpu.SemaphoreType.DMA,
            pltpu.SemaphoreType.DMA,
        ],
        compiler_params=pltpu.CompilerParams(
            use_tc_tiling_on_sc=True,
            disable_bounds_checks=True,
            skip_device_barrier=True,
        ),
    )
    def scale_kernel(emb_hbm, out_hbm, buf0, buf1, buf2,
                     si0, si1, si2, so0, so1, so2):
        wid = lax.axis_index("s") * info.num_cores + lax.axis_index("c")
        base = wid * rows_per_worker
        nbuf = 3
        bufs = (buf0, buf1, buf2)
        sin = (si0, si1, si2)
        sout = (so0, so1, so2)

        half = chunk_rows // 2

        def start_in(c, b):
            row = base + c * chunk_rows
            pltpu.async_copy(
                emb_hbm.at[pl.ds(row, half)],
                bufs[b].at[pl.ds(0, half)], sin[b])
            return pltpu.async_copy(
                emb_hbm.at[pl.ds(row + half, half)],
                bufs[b].at[pl.ds(half, half)], sin[b])

        in_copies = [None] * nbuf
        out_copies = [None] * nbuf
        for p in range(min(2, n_chunks)):
            in_copies[p] = start_in(p, p)
        for c in range(n_chunks):
            b = c % nbuf
            nb = (c + 2) % nbuf
            if c + 2 < n_chunks:
                if out_copies[nb] is not None:
                    out_copies[nb].wait()
                in_copies[nb] = start_in(c + 2, nb)
            in_copies[b].wait()
            in_copies[b].wait()
            buf = bufs[b]

            def row_body(r, _):
                @plsc.parallel_loop(0, dim, step=_LANES, unroll=8)
                def _scale(i):
                    buf[r, pl.ds(i, _LANES)] = buf[r, pl.ds(i, _LANES)] * scale

                return 0

            lax.fori_loop(0, chunk_rows, row_body, 0)

            out_copies[b] = pltpu.async_copy(
                buf, out_hbm.at[pl.ds(base + c * chunk_rows, chunk_rows)],
                sout[b])
        for p in range(min(nbuf, n_chunks)):
            out_copies[(n_chunks - 1 - p) % nbuf].wait()

    return scale_kernel


def kernel(x, emb_weight):
    seq_len = x.shape[1]
    dim = emb_weight.shape[1]
    scale = dim ** -0.5
    return _make_scale_kernel(seq_len, dim, scale)(emb_weight[:seq_len])


# speedup vs baseline: 1.5443x; 1.0463x over previous
import functools

import jax
import jax.numpy as jnp
from jax import lax
from jax.experimental import pallas as pl
from jax.experimental.pallas import tpu as pltpu
from jax.experimental.pallas import tpu_sc as plsc

_LANES = 16


def _chunk_schedule(rows_per_worker):
    if rows_per_worker < 96:
        assert rows_per_worker % 16 == 0
        sizes = [16, 16, 16]
        chunks = [(o, 16, o // 16 % 3) for o in range(0, rows_per_worker, 16)]
        return chunks, sizes
    sizes = [8, 8, 32, 32, 32]
    mid_rows = rows_per_worker - 32
    assert mid_rows % 32 == 0
    chunks = [(0, 8, 0), (8, 8, 1)]
    off = 16
    for i in range(mid_rows // 32):
        chunks.append((off, 32, 2 + i % 3))
        off += 32
    chunks += [(off, 8, 0), (off + 8, 8, 1)]
    return chunks, sizes


@functools.lru_cache(maxsize=None)
def _make_scale_kernel(rows: int, dim: int, scale: float):
    info = plsc.get_sparse_core_info()
    num_workers = info.num_cores * info.num_subcores
    assert rows % num_workers == 0
    rows_per_worker = rows // num_workers
    chunks, slot_sizes = _chunk_schedule(rows_per_worker)
    n_slots = len(slot_sizes)
    depth = 2

    mesh = plsc.VectorSubcoreMesh(core_axis_name="c", subcore_axis_name="s")

    scratch = [pltpu.VMEM((sz, dim), jnp.float32) for sz in slot_sizes]
    scratch += [pltpu.SemaphoreType.DMA] * (2 * n_slots)

    @functools.partial(
        pl.kernel,
        mesh=mesh,
        out_type=jax.ShapeDtypeStruct((rows, dim), jnp.float32),
        scratch_types=scratch,
        compiler_params=pltpu.CompilerParams(
            use_tc_tiling_on_sc=True,
            disable_bounds_checks=True,
            skip_device_barrier=True,
        ),
    )
    def scale_kernel(emb_hbm, out_hbm, *refs):
        bufs = refs[:n_slots]
        sin = refs[n_slots:2 * n_slots]
        sout = refs[2 * n_slots:3 * n_slots]
        wid = lax.axis_index("s") * info.num_cores + lax.axis_index("c")
        base = wid * rows_per_worker

        in_copies = {}
        out_copies = {}

        def start_in(ci):
            off, nr, slot = chunks[ci]
            if slot in out_copies:
                out_copies.pop(slot).wait()
            in_copies[slot] = pltpu.async_copy(
                emb_hbm.at[pl.ds(base + off, nr)], bufs[slot], sin[slot])

        for p in range(min(depth, len(chunks))):
            start_in(p)
        for ci, (off, nr, slot) in enumerate(chunks):
            if ci + depth < len(chunks):
                start_in(ci + depth)
            in_copies.pop(slot).wait()
            buf = bufs[slot]

            def row_body(r, _):
                @plsc.parallel_loop(0, dim, step=_LANES, unroll=8)
                def _scale(i):
                    buf[r, pl.ds(i, _LANES)] = buf[r, pl.ds(i, _LANES)] * scale

                return 0

            lax.fori_loop(0, nr, row_body, 0)

            out_copies[slot] = pltpu.async_copy(
                buf, out_hbm.at[pl.ds(base + off, nr)], sout[slot])
        for copy in out_copies.values():
            copy.wait()

    return scale_kernel


def kernel(x, emb_weight):
    seq_len = x.shape[1]
    dim = emb_weight.shape[1]
    scale = dim ** -0.5
    return _make_scale_kernel(seq_len, dim, scale)(emb_weight[:seq_len])
